# baseline (speedup 1.0000x reference)
import functools

import jax
import jax.numpy as jnp
from jax import lax
from jax.experimental import pallas as pl
from jax.experimental.pallas import tpu as pltpu
from jax.experimental.pallas import tpu_sc as plsc

B = 16384
D = 64
NEG = 20
NDOT = NEG + 1

NUM_CORES = 2
NUM_SUBCORES = 16
NW = NUM_CORES * NUM_SUBCORES

CB = 64
CHUNKS = B // (NW * CB)
ONC = CB * NDOT
TROWS = NDOT * 8 + 8
SCOL = 32


def _sc_body(c_hbm, on_hbm, win_hbm, wout_hbm, s_out,
             c_idxv, on_idxv, tiles, scores, red, srow,
             sem_a, sem_b):
  wid = lax.axis_index("s") * NUM_CORES + lax.axis_index("c")
  sems = (sem_a, sem_b)

  def fire(el, buf, sem):
    wc = c_idxv[pl.ds(el, 16)][0]
    pltpu.async_copy(
        win_hbm.at[pl.ds(pl.multiple_of((wc >> 3) * 8, 8), 8)],
        tiles.at[buf].at[pl.ds(NDOT * 8, 8)], sem)
    for t in range(NDOT):
      w = on_idxv[pl.ds(el * NDOT + t, 16)][0]
      pltpu.async_copy(
          wout_hbm.at[pl.ds(pl.multiple_of((w >> 3) * 8, 8), 8)],
          tiles.at[buf].at[pl.ds(t * 8, 8)], sem)

  def drain(buf, sem):
    for t in range(NDOT + 1):
      pltpu.make_async_copy(
          wout_hbm.at[pl.ds(0, 8)],
          tiles.at[buf].at[pl.ds(t * 8, 8)], sem).wait()

  def compute(el, buf):
    tb = tiles.at[buf]
    crow = NDOT * 8 + (c_idxv[pl.ds(el, 16)][0] & 7)
    c0 = tb[crow, pl.ds(0, 16)]
    c1 = tb[crow, pl.ds(16, 16)]
    c2 = tb[crow, pl.ds(32, 16)]
    c3 = tb[crow, pl.ds(48, 16)]
    for t in range(NDOT):
      r = t * 8 + (on_idxv[pl.ds(el * NDOT + t, 16)][0] & 7)
      q = (c0 * tb[r, pl.ds(0, 16)]
           + c1 * tb[r, pl.ds(16, 16)]
           + c2 * tb[r, pl.ds(32, 16)]
           + c3 * tb[r, pl.ds(48, 16)])
      rb = red.at[t]
      rb[pl.ds(0, 16)] = q
      q = q + rb[pl.ds(8, 16)]
      rb[pl.ds(0, 16)] = q
      q = q + rb[pl.ds(4, 16)]
      rb[pl.ds(0, 16)] = q
      q = q + rb[pl.ds(2, 16)]
      rb[pl.ds(0, 16)] = q
      q = q + rb[pl.ds(1, 16)]
      srow[pl.ds(t, 16)] = q
    scores[el, pl.ds(0, 16)] = srow[pl.ds(0, 16)]
    scores[el, pl.ds(16, 16)] = srow[pl.ds(16, 16)]

  def chunk_body(chunk, _):
    base = wid * (CHUNKS * CB) + chunk * CB
    pltpu.sync_copy(c_hbm.at[pl.ds(base, CB)], c_idxv.at[pl.ds(0, CB)])
    pltpu.sync_copy(on_hbm.at[pl.ds(base * NDOT, ONC)],
                    on_idxv.at[pl.ds(0, ONC)])

    fire(0, 0, sem_a)

    def pair_body(p, _):
      e0 = 2 * p
      fire(e0 + 1, 1, sem_b)
      drain(0, sem_a)
      compute(e0, 0)
      nxt = e0 + 2
      nxt = jnp.where(nxt >= CB, 0, nxt)
      fire(nxt, 0, sem_a)
      drain(1, sem_b)
      compute(e0 + 1, 1)
      return 0

    lax.fori_loop(0, CB // 2, pair_body, 0)
    drain(0, sem_a)

    pltpu.sync_copy(scores, s_out.at[pl.ds(base, CB)])
    return 0

  lax.fori_loop(0, CHUNKS, chunk_body, 0)


_sc_scores = functools.partial(
    pl.kernel,
    out_type=jax.ShapeDtypeStruct((B, SCOL), jnp.float32),
    mesh=plsc.VectorSubcoreMesh(core_axis_name="c", subcore_axis_name="s"),
    scratch_types=[
        pltpu.VMEM((CB + 16,), jnp.int32),
        pltpu.VMEM((ONC + 16,), jnp.int32),
        pltpu.VMEM((2, TROWS, D), jnp.float32),
        pltpu.VMEM((CB, SCOL), jnp.float32),
        pltpu.VMEM((NDOT, 32), jnp.float32),
        pltpu.VMEM((48,), jnp.float32),
        pltpu.SemaphoreType.DMA,
        pltpu.SemaphoreType.DMA,
    ],
)(_sc_body)


def _loss_body(s_ref, out_ref):
  def logsig(x):
    return -(jnp.maximum(-x, 0.0) + jnp.log1p(jnp.exp(-jnp.abs(x))))

  x = s_ref[...]
  col = lax.broadcasted_iota(jnp.int32, x.shape, 1) % SCOL
  pos_sum = jnp.sum(jnp.where(col == 0, logsig(x), 0.0))
  neg_sum = jnp.sum(jnp.where((col >= 1) & (col <= NEG), logsig(-x), 0.0))
  loss = -(pos_sum / B + neg_sum / (B * NEG))
  out_ref[...] = jnp.full((1, 1), loss, jnp.float32)


_loss_tc = pl.pallas_call(
    _loss_body,
    out_shape=jax.ShapeDtypeStruct((1, 1), jnp.float32),
)


def kernel(center_words, outside_words, negative_samples, W_in, W_out):
  c = center_words.astype(jnp.int32)
  on = jnp.concatenate(
      [outside_words.astype(jnp.int32)[:, None],
       negative_samples.astype(jnp.int32)], axis=1).reshape(B * NDOT)
  scores = _sc_scores(c, on, W_in, W_out)
  loss = _loss_tc(scores.reshape(B * SCOL // 128, 128))
  return loss[0, 0]


# speedup vs baseline: 1.1172x; 1.1172x over previous
import functools

import jax
import jax.numpy as jnp
from jax import lax
from jax.experimental import pallas as pl
from jax.experimental.pallas import tpu as pltpu
from jax.experimental.pallas import tpu_sc as plsc

B = 16384
D = 64
NEG = 20
NDOT = NEG + 1

NUM_CORES = 2
NUM_SUBCORES = 16
NW = NUM_CORES * NUM_SUBCORES

CB = 64
CHUNKS = B // (NW * CB)
ONC = CB * NDOT
TROWS = NDOT * 8 + 8
SCOL = 32


def _sc_body(c_hbm, on_hbm, win_hbm, wout_hbm, s_out,
             c_idxv, on_idxv, smc, smon, tiles, scores, red, srow,
             sem_a, sem_b):
  wid = lax.axis_index("s") * NUM_CORES + lax.axis_index("c")
  sems = (sem_a, sem_b)

  def fire(el, buf, sem):
    wc = smc[el]
    pltpu.async_copy(
        win_hbm.at[pl.ds(pl.multiple_of((wc >> 3) * 8, 8), 8)],
        tiles.at[buf].at[pl.ds(NDOT * 8, 8)], sem)
    for t in range(NDOT):
      w = smon[el * NDOT + t]
      pltpu.async_copy(
          wout_hbm.at[pl.ds(pl.multiple_of((w >> 3) * 8, 8), 8)],
          tiles.at[buf].at[pl.ds(t * 8, 8)], sem)

  def drain(buf, sem):
    for t in range(NDOT + 1):
      pltpu.make_async_copy(
          wout_hbm.at[pl.ds(0, 8)],
          tiles.at[buf].at[pl.ds(t * 8, 8)], sem).wait()

  def compute(el, buf):
    tb = tiles.at[buf]
    crow = NDOT * 8 + (smc[el] & 7)
    c0 = tb[crow, pl.ds(0, 16)]
    c1 = tb[crow, pl.ds(16, 16)]
    c2 = tb[crow, pl.ds(32, 16)]
    c3 = tb[crow, pl.ds(48, 16)]
    for t in range(NDOT):
      r = t * 8 + (smon[el * NDOT + t] & 7)
      q = (c0 * tb[r, pl.ds(0, 16)]
           + c1 * tb[r, pl.ds(16, 16)]
           + c2 * tb[r, pl.ds(32, 16)]
           + c3 * tb[r, pl.ds(48, 16)])
      rb = red.at[t]
      rb[pl.ds(0, 16)] = q
      q = q + rb[pl.ds(8, 16)]
      rb[pl.ds(0, 16)] = q
      q = q + rb[pl.ds(4, 16)]
      rb[pl.ds(0, 16)] = q
      q = q + rb[pl.ds(2, 16)]
      rb[pl.ds(0, 16)] = q
      q = q + rb[pl.ds(1, 16)]
      srow[pl.ds(t, 16)] = q
    scores[el, pl.ds(0, 16)] = srow[pl.ds(0, 16)]
    scores[el, pl.ds(16, 16)] = srow[pl.ds(16, 16)]

  def chunk_body(chunk, _):
    base = wid * (CHUNKS * CB) + chunk * CB
    pltpu.sync_copy(c_hbm.at[pl.ds(base, CB)], c_idxv)
    pltpu.sync_copy(on_hbm.at[pl.ds(base * NDOT, ONC)], on_idxv)
    for g in range(CB // 16):
      v = c_idxv[pl.ds(g * 16, 16)]
      for j in range(16):
        smc[g * 16 + j] = v[j]
    for g in range(ONC // 16):
      v = on_idxv[pl.ds(g * 16, 16)]
      for j in range(16):
        smon[g * 16 + j] = v[j]

    fire(0, 0, sem_a)

    def pair_body(p, _):
      e0 = 2 * p
      fire(e0 + 1, 1, sem_b)
      drain(0, sem_a)
      compute(e0, 0)
      nxt = e0 + 2
      nxt = jnp.where(nxt >= CB, 0, nxt)
      fire(nxt, 0, sem_a)
      drain(1, sem_b)
      compute(e0 + 1, 1)
      return 0

    lax.fori_loop(0, CB // 2, pair_body, 0)
    drain(0, sem_a)

    pltpu.sync_copy(scores, s_out.at[pl.ds(base, CB)])
    return 0

  lax.fori_loop(0, CHUNKS, chunk_body, 0)


_sc_scores = functools.partial(
    pl.kernel,
    out_type=jax.ShapeDtypeStruct((B, SCOL), jnp.float32),
    mesh=plsc.VectorSubcoreMesh(core_axis_name="c", subcore_axis_name="s"),
    scratch_types=[
        pltpu.VMEM((CB,), jnp.int32),
        pltpu.VMEM((ONC,), jnp.int32),
        pltpu.SMEM((CB,), jnp.int32),
        pltpu.SMEM((ONC,), jnp.int32),
        pltpu.VMEM((2, TROWS, D), jnp.float32),
        pltpu.VMEM((CB, SCOL), jnp.float32),
        pltpu.VMEM((NDOT, 32), jnp.float32),
        pltpu.VMEM((48,), jnp.float32),
        pltpu.SemaphoreType.DMA,
        pltpu.SemaphoreType.DMA,
    ],
)(_sc_body)


def _loss_body(s_ref, out_ref):
  def logsig(x):
    return -(jnp.maximum(-x, 0.0) + jnp.log1p(jnp.exp(-jnp.abs(x))))

  x = s_ref[...]
  col = lax.broadcasted_iota(jnp.int32, x.shape, 1) % SCOL
  pos_sum = jnp.sum(jnp.where(col == 0, logsig(x), 0.0))
  neg_sum = jnp.sum(jnp.where((col >= 1) & (col <= NEG), logsig(-x), 0.0))
  loss = -(pos_sum / B + neg_sum / (B * NEG))
  out_ref[...] = jnp.full((1, 1), loss, jnp.float32)


_loss_tc = pl.pallas_call(
    _loss_body,
    out_shape=jax.ShapeDtypeStruct((1, 1), jnp.float32),
)


def kernel(center_words, outside_words, negative_samples, W_in, W_out):
  c = center_words.astype(jnp.int32)
  on = jnp.concatenate(
      [outside_words.astype(jnp.int32)[:, None],
       negative_samples.astype(jnp.int32)], axis=1).reshape(B * NDOT)
  scores = _sc_scores(c, on, W_in, W_out)
  loss = _loss_tc(scores.reshape(B * SCOL // 128, 128))
  return loss[0, 0]
